# baseline (speedup 1.0000x reference)
import jax
import jax.numpy as jnp
from jax import lax
from jax.experimental import pallas as pl

_TILE = 5000


def _tpl_kernel(x_ref, bw_ref, o_ref):
    f32 = jnp.float32
    F = 8
    C = 3
    B = 4 * C
    K = F * B
    D = x_ref.shape[1]

    xt = x_ref[...]
    h = jnp.maximum(
        jnp.dot(xt, bw_ref[224:352, :], preferred_element_type=f32)
        + bw_ref[480:481, :], 0.0)
    ge = lax.broadcasted_iota(jnp.int32, (D, K), 0)
    gc = lax.broadcasted_iota(jnp.int32, (D, K), 1)
    G = (gc // B == ge).astype(f32)
    W2G = jnp.dot(bw_ref[352:480, :], G, preferred_element_type=f32)
    b2G = jnp.dot(bw_ref[481:482, :], G, preferred_element_type=f32)
    v96 = jnp.dot(h, W2G, preferred_element_type=f32) + b2G

    col = lax.broadcasted_iota(jnp.int32, (1, K), 1)
    k = col % B
    j = k % C

    p = bw_ref[483:484, :]

    def sel3(o):
        return jnp.where(j == 0, p[0:1, o:o + 1],
                         jnp.where(j == 1, p[0:1, o + 1:o + 2],
                                   p[0:1, o + 2:o + 3]))

    t96 = sel3(0)
    mu0 = sel3(3)
    mu1 = sel3(6)
    lw = sel3(9) + sel3(12)
    lb96 = sel3(15)
    c0 = sel3(18)
    c1 = sel3(21)
    s = p[0:1, 24:25]
    inv2s = 1.0 / (2.0 * s * s)
    absr = jnp.abs(p[0:1, 25:26])
    gP = -(inv2s + inv2s)
    gQ = (inv2s + inv2s) * (mu0 + mu1)
    gR = -inv2s * (mu0 * mu0 + mu1 * mu1)

    tri = jnp.maximum(jnp.minimum(v96 + v96 - t96, t96), 0.0)
    gau = jnp.exp(v96 * (gP * v96 + gQ) + gR)
    lin = v96 * lw + lb96
    d1 = jnp.abs(v96 - c0) + jnp.abs(v96 - c1)
    rat = 1.0 / (1.0 + d1) - 1.0 / (1.0 + jnp.abs(absr - d1))
    coord = jnp.where(k < C, tri,
                      jnp.where(k < 2 * C, gau,
                                jnp.where(k < 3 * C, lin, rat)))

    acc = (jnp.dot(xt, bw_ref[0:128, :], preferred_element_type=f32)
           + jnp.dot(coord, bw_ref[128:224, :], preferred_element_type=f32)
           + bw_ref[482:483, :])
    o_ref[...] = jnp.maximum(acc, 0.0)


def kernel(x, edge_index, W1, b1, W2, b2, t_param, gauss_mu, gauss_sigma,
           line_W, line_b, rat_c, rat_r, out_W, out_b):
    del edge_index
    N, D = x.shape
    f32 = jnp.float32

    z = lambda n: jnp.zeros((n,), f32)
    sm = jnp.concatenate([
        b1, z(128 - b1.shape[0]),
        b2, z(128 - b2.shape[0]),
        out_b,
        t_param, gauss_mu[:, 0], gauss_mu[:, 1],
        line_W[:, 0], line_W[:, 1], line_b,
        rat_c[:, 0], rat_c[:, 1],
        gauss_sigma[None], rat_r[None], z(102),
    ]).reshape(4, 128)
    bw = jnp.concatenate([
        out_W,
        jnp.pad(W1, ((0, 0), (0, 128 - W1.shape[1]))),
        jnp.pad(W2, ((0, 128 - W2.shape[0]), (0, 128 - W2.shape[1]))),
        sm,
    ], axis=0)

    grid = (N // _TILE,)
    out = pl.pallas_call(
        _tpl_kernel,
        grid=grid,
        in_specs=[
            pl.BlockSpec((_TILE, D), lambda i: (i, 0)),
            pl.BlockSpec(bw.shape, lambda i: (0, 0)),
        ],
        out_specs=pl.BlockSpec((_TILE, out_W.shape[1]), lambda i: (i, 0)),
        out_shape=jax.ShapeDtypeStruct((N, out_W.shape[1]), f32),
    )(x, bw)
    return out


# speedup vs baseline: 1.0327x; 1.0327x over previous
import jax
import jax.numpy as jnp
from jax import lax
from jax.experimental import pallas as pl

_TILE = 5000


def _tpl_kernel(x_ref, bw_ref, o_ref):
    f32 = jnp.float32
    F = 8
    C = 3
    B = 4 * C
    K = F * B
    D = x_ref.shape[1]

    xt = x_ref[...]
    xb = xt.astype(jnp.bfloat16)
    h = jnp.maximum(
        jnp.dot(xb, bw_ref[224:352, :].astype(jnp.bfloat16),
                preferred_element_type=f32)
        + bw_ref[480:481, :], 0.0)
    v = jnp.dot(h, bw_ref[352:480, :], preferred_element_type=f32) \
        + bw_ref[481:482, :]

    ge = lax.broadcasted_iota(jnp.int32, (D, K), 0)
    gc = lax.broadcasted_iota(jnp.int32, (D, K), 1)
    G = (gc // B == ge).astype(f32)
    v96 = jnp.dot(v, G, preferred_element_type=f32)

    col = lax.broadcasted_iota(jnp.int32, (1, K), 1)
    k = col % B
    j = k % C

    p = bw_ref[483:484, :]

    def sel3(o):
        return jnp.where(j == 0, p[0:1, o:o + 1],
                         jnp.where(j == 1, p[0:1, o + 1:o + 2],
                                   p[0:1, o + 2:o + 3]))

    t96 = sel3(0)
    mu0 = sel3(3)
    mu1 = sel3(6)
    lw = sel3(9) + sel3(12)
    lb96 = sel3(15)
    c0 = sel3(18)
    c1 = sel3(21)
    s = p[0:1, 24:25]
    inv2s = 1.0 / (2.0 * s * s)
    absr = jnp.abs(p[0:1, 25:26])
    gP = -(inv2s + inv2s)
    gQ = (inv2s + inv2s) * (mu0 + mu1)
    gR = -inv2s * (mu0 * mu0 + mu1 * mu1)

    tri = jnp.maximum(jnp.minimum(v96 + v96 - t96, t96), 0.0)
    gau = jnp.exp(v96 * (gP * v96 + gQ) + gR)
    lin = v96 * lw + lb96
    d1 = jnp.abs(v96 - c0) + jnp.abs(v96 - c1)
    rat = 1.0 / (1.0 + d1) - 1.0 / (1.0 + jnp.abs(absr - d1))
    coord = jnp.where(k < C, tri,
                      jnp.where(k < 2 * C, gau,
                                jnp.where(k < 3 * C, lin, rat)))

    acc = (jnp.dot(xb, bw_ref[0:128, :].astype(jnp.bfloat16),
                   preferred_element_type=f32)
           + jnp.dot(coord, bw_ref[128:224, :], preferred_element_type=f32)
           + bw_ref[482:483, :])
    o_ref[...] = jnp.maximum(acc, 0.0)


def kernel(x, edge_index, W1, b1, W2, b2, t_param, gauss_mu, gauss_sigma,
           line_W, line_b, rat_c, rat_r, out_W, out_b):
    del edge_index
    N, D = x.shape
    f32 = jnp.float32

    z = lambda n: jnp.zeros((n,), f32)
    sm = jnp.concatenate([
        b1, z(128 - b1.shape[0]),
        b2, z(128 - b2.shape[0]),
        out_b,
        t_param, gauss_mu[:, 0], gauss_mu[:, 1],
        line_W[:, 0], line_W[:, 1], line_b,
        rat_c[:, 0], rat_c[:, 1],
        gauss_sigma[None], rat_r[None], z(102),
    ]).reshape(4, 128)
    bw = jnp.concatenate([
        out_W,
        jnp.pad(W1, ((0, 0), (0, 128 - W1.shape[1]))),
        jnp.pad(W2, ((0, 128 - W2.shape[0]), (0, 128 - W2.shape[1]))),
        sm,
    ], axis=0)

    grid = (N // _TILE,)
    out = pl.pallas_call(
        _tpl_kernel,
        grid=grid,
        in_specs=[
            pl.BlockSpec((_TILE, D), lambda i: (i, 0)),
            pl.BlockSpec(bw.shape, lambda i: (0, 0)),
        ],
        out_specs=pl.BlockSpec((_TILE, out_W.shape[1]), lambda i: (i, 0)),
        out_shape=jax.ShapeDtypeStruct((N, out_W.shape[1]), f32),
    )(x, bw)
    return out


# speedup vs baseline: 1.1428x; 1.1067x over previous
import jax
import jax.numpy as jnp
from jax import lax
from jax.experimental import pallas as pl

_TILE = 2504


def _tpl_kernel(x_ref, bw_ref, o_ref):
    f32 = jnp.float32
    F = 8
    C = 3
    B = 4 * C
    K = F * B
    D = x_ref.shape[1]

    xt = x_ref[...]
    h = jnp.maximum(
        jnp.dot(xt, bw_ref[224:352, :], preferred_element_type=f32)
        + bw_ref[480:481, :], 0.0)
    v = jnp.dot(h, bw_ref[352:480, :], preferred_element_type=f32) \
        + bw_ref[481:482, :]

    ge = lax.broadcasted_iota(jnp.int32, (D, K), 0)
    gc = lax.broadcasted_iota(jnp.int32, (D, K), 1)
    G = (gc // B == ge).astype(f32)
    v96 = jnp.dot(v, G, preferred_element_type=f32)

    col = lax.broadcasted_iota(jnp.int32, (1, K), 1)
    k = col % B
    j = k % C

    p = bw_ref[483:484, :]

    def sel3(o):
        return jnp.where(j == 0, p[0:1, o:o + 1],
                         jnp.where(j == 1, p[0:1, o + 1:o + 2],
                                   p[0:1, o + 2:o + 3]))

    t96 = sel3(0)
    mu0 = sel3(3)
    mu1 = sel3(6)
    lw = sel3(9) + sel3(12)
    lb96 = sel3(15)
    c0 = sel3(18)
    c1 = sel3(21)
    s = p[0:1, 24:25]
    inv2s = 1.0 / (2.0 * s * s)
    absr = jnp.abs(p[0:1, 25:26])
    gP = -(inv2s + inv2s)
    gQ = (inv2s + inv2s) * (mu0 + mu1)
    gR = -inv2s * (mu0 * mu0 + mu1 * mu1)

    tri = jnp.maximum(jnp.minimum(v96 + v96 - t96, t96), 0.0)
    gau = jnp.exp(v96 * (gP * v96 + gQ) + gR)
    lin = v96 * lw + lb96
    d1 = jnp.abs(v96 - c0) + jnp.abs(v96 - c1)
    rat = 1.0 / (1.0 + d1) - 1.0 / (1.0 + jnp.abs(absr - d1))
    coord = jnp.where(k < C, tri,
                      jnp.where(k < 2 * C, gau,
                                jnp.where(k < 3 * C, lin, rat)))

    acc = (jnp.dot(xt, bw_ref[0:128, :], preferred_element_type=f32)
           + jnp.dot(coord, bw_ref[128:224, :], preferred_element_type=f32)
           + bw_ref[482:483, :])
    o_ref[...] = jnp.maximum(acc, 0.0)


def kernel(x, edge_index, W1, b1, W2, b2, t_param, gauss_mu, gauss_sigma,
           line_W, line_b, rat_c, rat_r, out_W, out_b):
    del edge_index
    N, D = x.shape
    f32 = jnp.float32

    z = lambda n: jnp.zeros((n,), f32)
    sm = jnp.concatenate([
        b1, z(128 - b1.shape[0]),
        b2, z(128 - b2.shape[0]),
        out_b,
        t_param, gauss_mu[:, 0], gauss_mu[:, 1],
        line_W[:, 0], line_W[:, 1], line_b,
        rat_c[:, 0], rat_c[:, 1],
        gauss_sigma[None], rat_r[None], z(102),
    ]).reshape(4, 128)
    bw = jnp.concatenate([
        out_W,
        jnp.pad(W1, ((0, 0), (0, 128 - W1.shape[1]))),
        jnp.pad(W2, ((0, 128 - W2.shape[0]), (0, 128 - W2.shape[1]))),
        sm,
    ], axis=0)

    grid = ((N + _TILE - 1) // _TILE,)
    out = pl.pallas_call(
        _tpl_kernel,
        grid=grid,
        in_specs=[
            pl.BlockSpec((_TILE, D), lambda i: (i, 0)),
            pl.BlockSpec(bw.shape, lambda i: (0, 0)),
        ],
        out_specs=pl.BlockSpec((_TILE, out_W.shape[1]), lambda i: (i, 0)),
        out_shape=jax.ShapeDtypeStruct((N, out_W.shape[1]), f32),
    )(x, bw)
    return out


# speedup vs baseline: 1.1477x; 1.0043x over previous
import jax
import jax.numpy as jnp
from jax import lax
from jax.experimental import pallas as pl

_TILE = 3336


def _tpl_kernel(x_ref, bw_ref, o_ref):
    f32 = jnp.float32
    F = 8
    C = 3
    B = 4 * C
    K = F * B
    D = x_ref.shape[1]

    xt = x_ref[...]
    h = jnp.maximum(
        jnp.dot(xt, bw_ref[224:352, :], preferred_element_type=f32)
        + bw_ref[480:481, :], 0.0)
    v = jnp.dot(h, bw_ref[352:480, :], preferred_element_type=f32) \
        + bw_ref[481:482, :]

    ge = lax.broadcasted_iota(jnp.int32, (D, K), 0)
    gc = lax.broadcasted_iota(jnp.int32, (D, K), 1)
    G = (gc // B == ge).astype(f32)
    v96 = jnp.dot(v, G, preferred_element_type=f32)

    col = lax.broadcasted_iota(jnp.int32, (1, K), 1)
    k = col % B
    j = k % C

    p = bw_ref[483:484, :]

    def sel3(o):
        return jnp.where(j == 0, p[0:1, o:o + 1],
                         jnp.where(j == 1, p[0:1, o + 1:o + 2],
                                   p[0:1, o + 2:o + 3]))

    t96 = sel3(0)
    mu0 = sel3(3)
    mu1 = sel3(6)
    lw = sel3(9) + sel3(12)
    lb96 = sel3(15)
    c0 = sel3(18)
    c1 = sel3(21)
    s = p[0:1, 24:25]
    inv2s = 1.0 / (2.0 * s * s)
    absr = jnp.abs(p[0:1, 25:26])
    gP = -(inv2s + inv2s)
    gQ = (inv2s + inv2s) * (mu0 + mu1)
    gR = -inv2s * (mu0 * mu0 + mu1 * mu1)

    tri = jnp.maximum(jnp.minimum(v96 + v96 - t96, t96), 0.0)
    gau = jnp.exp(v96 * (gP * v96 + gQ) + gR)
    lin = v96 * lw + lb96
    d1 = jnp.abs(v96 - c0) + jnp.abs(v96 - c1)
    rat = 1.0 / (1.0 + d1) - 1.0 / (1.0 + jnp.abs(absr - d1))
    coord = jnp.where(k < C, tri,
                      jnp.where(k < 2 * C, gau,
                                jnp.where(k < 3 * C, lin, rat)))

    acc = (jnp.dot(xt, bw_ref[0:128, :], preferred_element_type=f32)
           + jnp.dot(coord, bw_ref[128:224, :], preferred_element_type=f32)
           + bw_ref[482:483, :])
    o_ref[...] = jnp.maximum(acc, 0.0)


def kernel(x, edge_index, W1, b1, W2, b2, t_param, gauss_mu, gauss_sigma,
           line_W, line_b, rat_c, rat_r, out_W, out_b):
    del edge_index
    N, D = x.shape
    f32 = jnp.float32

    z = lambda n: jnp.zeros((n,), f32)
    sm = jnp.concatenate([
        b1, z(128 - b1.shape[0]),
        b2, z(128 - b2.shape[0]),
        out_b,
        t_param, gauss_mu[:, 0], gauss_mu[:, 1],
        line_W[:, 0], line_W[:, 1], line_b,
        rat_c[:, 0], rat_c[:, 1],
        gauss_sigma[None], rat_r[None], z(102),
    ]).reshape(4, 128)
    bw = jnp.concatenate([
        out_W,
        jnp.pad(W1, ((0, 0), (0, 128 - W1.shape[1]))),
        jnp.pad(W2, ((0, 128 - W2.shape[0]), (0, 128 - W2.shape[1]))),
        sm,
    ], axis=0)

    grid = ((N + _TILE - 1) // _TILE,)
    out = pl.pallas_call(
        _tpl_kernel,
        grid=grid,
        in_specs=[
            pl.BlockSpec((_TILE, D), lambda i: (i, 0)),
            pl.BlockSpec(bw.shape, lambda i: (0, 0)),
        ],
        out_specs=pl.BlockSpec((_TILE, out_W.shape[1]), lambda i: (i, 0)),
        out_shape=jax.ShapeDtypeStruct((N, out_W.shape[1]), f32),
    )(x, bw)
    return out
